# Initial kernel scaffold; baseline (speedup 1.0000x reference)
import jax
import jax.numpy as jnp
from jax.experimental import pallas as pl


def kernel(points, pose, extrinsic, intrinsic):
    raise NotImplementedError("write your pallas kernel here")


# speedup vs baseline: 225.6439x; 225.6439x over previous
import jax
import jax.numpy as jnp
from jax.experimental import pallas as pl
from jax.experimental.pallas import tpu as pltpu

_H, _W = 256, 384
_OH, _OW = 85, 128
_N = 8192
_BIG = 1e30


def _edt_kernel(ptsT_ref, par_ref, ah_ref, awt_ref, out_ref,
                depth_s, g1_s, v1_s, l1_s):
    x = ptsT_ref[0, 0:1, :]
    y = ptsT_ref[0, 1:2, :]
    z = ptsT_ref[0, 2:3, :]
    p = lambda k: par_ref[0, 0, k]
    bf = lambda t: t.astype(jnp.bfloat16).astype(jnp.float32)
    bx, by, bz = bf(x), bf(y), bf(z)
    pw0 = (p(0) * bx + p(1) * by) + p(2) * bz + p(9)
    pw1 = (p(3) * bx + p(4) * by) + p(5) * bz + p(10)
    pw2 = (p(6) * bx + p(7) * by) + p(8) * bz + p(11)
    b0, b1, b2 = bf(pw0), bf(pw1), bf(pw2)
    cam0 = ((p(12) * b0 + p(13) * b1) + p(14) * b2) + p(15)
    cam1 = ((p(16) * b0 + p(17) * b1) + p(18) * b2) + p(19)
    cam2 = ((p(20) * b0 + p(21) * b1) + p(22) * b2) + p(23)
    zc = jnp.where(jnp.abs(cam2) < 1e-6, jnp.float32(1e-6), cam2)
    u = p(24) * cam0 / zc + p(26)
    v = p(25) * cam1 / zc + p(27)
    ri = jnp.floor(v)
    ci = jnp.floor(u)
    dv = jnp.sqrt(x * x + y * y + z * z)
    m = (ri >= 0) & (ri < _H) & (ci >= 0) & (ci < _W)
    val = jnp.where(m, dv, jnp.float32(0.0))

    iota_r = jax.lax.broadcasted_iota(jnp.int32, (_H, _N), 0).astype(jnp.float32)
    iota_c = jax.lax.broadcasted_iota(jnp.int32, (_W, _N), 0).astype(jnp.float32)
    a_oh = (iota_r == ri).astype(jnp.float32)
    bct = jnp.where(iota_c == ci, val, jnp.float32(0.0))
    depth = jax.lax.dot_general(
        a_oh, bct, (((1,), (1,)), ((), ())),
        preferred_element_type=jnp.float32)

    valid = depth != 0

    r2d = jax.lax.broadcasted_iota(jnp.int32, (_H, _W), 0).astype(jnp.float32)
    c2d = jax.lax.broadcasted_iota(jnp.int32, (_H, _W), 1).astype(jnp.float32)
    rmin = jnp.min(jnp.where(valid, r2d, jnp.float32(_H))).astype(jnp.int32)
    rmax = jnp.max(jnp.where(valid, r2d, jnp.float32(-1))).astype(jnp.int32)
    cmin = jnp.min(jnp.where(valid, c2d, jnp.float32(_W))).astype(jnp.int32)
    cmax = jnp.max(jnp.where(valid, c2d, jnp.float32(-1))).astype(jnp.int32)

    depth_s[...] = depth

    def p1_body(rp, carry):
        bd2, br, bv = carry
        rowd = depth_s[pl.ds(rp, 1), :]
        pen = jnp.where(rowd != 0, jnp.float32(0.0), jnp.float32(_BIG))
        rpf = rp.astype(jnp.float32)
        dr = r2d[:, 0:1] - rpf
        cand = dr * dr + pen
        upd = cand < bd2
        bd2 = jnp.where(upd, cand, bd2)
        br = jnp.where(upd, rpf, br)
        bv = jnp.where(upd, rowd, bv)
        return bd2, br, bv

    init = (jnp.full((_H, _W), _BIG, jnp.float32),
            jnp.zeros((_H, _W), jnp.float32),
            jnp.zeros((_H, _W), jnp.float32))
    g1, r1, v1 = jax.lax.fori_loop(rmin, rmax + 1, p1_body, init)
    lin1 = r1 * jnp.float32(_W) + c2d

    iw = jax.lax.broadcasted_iota(jnp.int32, (_W, _W), 0)
    jw = jax.lax.broadcasted_iota(jnp.int32, (_W, _W), 1)
    eye_w = (iw == jw).astype(jnp.float32)
    nt = (((1,), (1,)), ((), ()))

    def tr(x):
        return jax.lax.dot_general(eye_w, x, nt,
                                   preferred_element_type=jnp.float32)

    g1_s[...] = tr(g1)
    v1_s[...] = tr(v1)
    depth_t = tr(depth)
    l1_s[...] = tr(lin1)

    c2dt = jax.lax.broadcasted_iota(jnp.int32, (_W, _H), 0).astype(jnp.float32)

    def p2_body(cp, carry):
        bd2, bl, bv = carry
        rowg = g1_s[pl.ds(cp, 1), :]
        rowl = l1_s[pl.ds(cp, 1), :]
        rowv = v1_s[pl.ds(cp, 1), :]
        cpf = cp.astype(jnp.float32)
        dc = c2dt[:, 0:1] - cpf
        cand = rowg + dc * dc
        upd = (cand < bd2) | ((cand == bd2) & (rowl < bl))
        bd2 = jnp.where(upd, cand, bd2)
        bl = jnp.where(upd, rowl, bl)
        bv = jnp.where(upd, rowv, bv)
        return bd2, bl, bv

    init2 = (jnp.full((_W, _H), _BIG, jnp.float32),
             jnp.full((_W, _H), _BIG, jnp.float32),
             jnp.zeros((_W, _H), jnp.float32))
    d2t, _, nnvt = jax.lax.fori_loop(cmin, cmax + 1, p2_body, init2)

    valid_t = depth_t != 0
    dist_t = jnp.where(valid_t, jnp.float32(0.0), jnp.sqrt(d2t))
    filled_t = jnp.where(valid_t, depth_t, nnvt)

    ah = ah_ref[...]
    awt = awt_ref[...]
    t1 = jax.lax.dot_general(ah, filled_t, nt,
                             preferred_element_type=jnp.float32)
    t2 = jax.lax.dot_general(ah, dist_t, nt,
                             preferred_element_type=jnp.float32)
    out_ref[0, 0] = jnp.dot(t1, awt, preferred_element_type=jnp.float32)
    out_ref[0, 1] = jnp.dot(t2, awt, preferred_element_type=jnp.float32)


def _resize_mats():
    ah = jax.image.resize(jnp.eye(_H, dtype=jnp.float32), (_OH, _H), "bilinear")
    aw = jax.image.resize(jnp.eye(_W, dtype=jnp.float32), (_OW, _W), "bilinear")
    return ah, aw.T


def kernel(points, pose, extrinsic, intrinsic):
    B = points.shape[0]
    ptsT = jnp.swapaxes(points, 1, 2)
    bfr = lambda t: t.astype(jnp.bfloat16).astype(jnp.float32)
    Rp = bfr(pose[:, :3, :3])
    tp = pose[:, :3, 3]
    Ee = bfr(extrinsic[:, :3, :4])
    par = jnp.concatenate([
        Rp.reshape(B, 9), tp, Ee.reshape(B, 12),
        intrinsic[:, 0, 0:1], intrinsic[:, 1, 1:2],
        intrinsic[:, 0, 2:3], intrinsic[:, 1, 2:3],
    ], axis=1).reshape(B, 1, 28)
    ah, awt = _resize_mats()

    out = pl.pallas_call(
        _edt_kernel,
        grid=(B,),
        in_specs=[
            pl.BlockSpec((1, 3, _N), lambda i: (i, 0, 0)),
            pl.BlockSpec((1, 1, 28), lambda i: (i, 0, 0), memory_space=pltpu.SMEM),
            pl.BlockSpec((_OH, _H), lambda i: (0, 0)),
            pl.BlockSpec((_W, _OW), lambda i: (0, 0)),
        ],
        out_specs=pl.BlockSpec((1, 2, _OH, _OW), lambda i: (i, 0, 0, 0)),
        out_shape=jax.ShapeDtypeStruct((B, 2, _OH, _OW), jnp.float32),
        scratch_shapes=[pltpu.VMEM((_H, _W), jnp.float32)]
        + [pltpu.VMEM((_W, _H), jnp.float32) for _ in range(3)],
        compiler_params=pltpu.CompilerParams(
            dimension_semantics=("parallel",)),
    )(ptsT, par, ah, awt)
    return out
